# baseline (speedup 1.0000x reference)
import functools

import jax
import jax.numpy as jnp
import numpy as np
from jax import lax
from jax.experimental import pallas as pl
from jax.experimental.pallas import tpu as pltpu
from jax.experimental.pallas import tpu_sc as plsc

_INFO = plsc.get_sparse_core_info()
_NC, _NS, _L = _INFO.num_cores, _INFO.num_subcores, _INFO.num_lanes
_NW = _NC * _NS

_CP = 32


@functools.partial(jax.jit, static_argnums=(0, 1, 2, 3, 4))
def _tokenize(B, F, NCAT, V, D, xnum_f, xcat_f, w, b, cls, coffp, table):
    T = 1 + F + NCAT
    HEAD = 1 + F
    FP = _L
    CH = 8
    PER_W = B // _NW
    NCH = PER_W // CH
    NVEC = D // _L
    CATN = CH * NCAT
    CATP = CATN + _L
    IDXN = CH * _CP
    XNN = CH * F
    XNP = XNN + _L

    mesh = plsc.VectorSubcoreMesh(core_axis_name="c", subcore_axis_name="s")

    @functools.partial(
        pl.kernel,
        out_type=jax.ShapeDtypeStruct((B * T, D), jnp.float32),
        mesh=mesh,
        scratch_types=[
            pltpu.VMEM((CATP,), jnp.int32),
            pltpu.VMEM((CATP,), jnp.int32),
            pltpu.VMEM((IDXN,), jnp.int32),
            pltpu.VMEM((IDXN,), jnp.int32),
            pltpu.VMEM((XNP,), jnp.float32),
            pltpu.VMEM((XNP,), jnp.float32),
            pltpu.VMEM((CH * T, D), jnp.float32),
            pltpu.VMEM((CH * T, D), jnp.float32),
            pltpu.VMEM((D,), jnp.float32),
            pltpu.VMEM((D,), jnp.float32),
            pltpu.VMEM((D,), jnp.float32),
            pltpu.VMEM((2 * _L,), jnp.int32),
            pltpu.SemaphoreType.DMA,
            pltpu.SemaphoreType.DMA,
            pltpu.SemaphoreType.DMA,
            pltpu.SemaphoreType.DMA,
            pltpu.SemaphoreType.DMA,
        ],
    )
    def tok(xnum_hbm, xcat_hbm, w_hbm, b_hbm, cls_hbm, coffp_hbm,
            table_hbm, out_hbm,
            rawa_v, rawb_v, idxa_v, idxb_v, xna_v, xnb_v,
            blka_v, blkb_v, w_v, b_v, c_v, coffp_v,
            gsem, wsema, wsemb, isema, isemb):
        wid = lax.axis_index("s") * _NC + lax.axis_index("c")
        base = wid * PER_W
        pltpu.sync_copy(w_hbm, w_v)
        pltpu.sync_copy(b_hbm, b_v)
        pltpu.sync_copy(cls_hbm, c_v)
        pltpu.sync_copy(coffp_hbm, coffp_v)
        w_r = [w_v[pl.ds(d * _L, _L)] for d in range(NVEC)]
        b_r = [b_v[pl.ds(d * _L, _L)] for d in range(NVEC)]
        c_r = [c_v[pl.ds(d * _L, _L)] for d in range(NVEC)]
        co0 = coffp_v[pl.ds(0, _L)]
        co1 = coffp_v[pl.ds(_L, _L)]
        lanes = jnp.arange(_L, dtype=jnp.int32)

        def fetch(b0, raw_v, xn_v, isem):
            pltpu.async_copy(xcat_hbm.at[pl.ds(b0 * NCAT, CATN)],
                             raw_v.at[pl.ds(0, CATN)], isem)
            pltpu.async_copy(xnum_hbm.at[pl.ds(b0 * F, XNN)],
                             xn_v.at[pl.ds(0, XNN)], isem)

        def drain_in(raw_v, xn_v, isem):
            pltpu.make_async_copy(
                xcat_hbm.at[pl.ds(0, CATN)], raw_v.at[pl.ds(0, CATN)],
                isem).wait()
            pltpu.make_async_copy(
                xnum_hbm.at[pl.ds(0, XNN)], xn_v.at[pl.ds(0, XNN)],
                isem).wait()

        def drain_out(blk_v, wsem):
            pltpu.make_async_copy(
                out_hbm.at[pl.ds(0, CH * T)], blk_v, wsem).wait()

        def do_chunk(b0, raw_v, idx_v, xn_v, blk_v, wsem, isem):
            drain_in(raw_v, xn_v, isem)
            for bi in range(CH):
                v0 = raw_v[pl.ds(bi * NCAT, _L)]
                v1 = raw_v[pl.ds(bi * NCAT + _L, _L)]
                idx_v[pl.ds(bi * _CP, _L)] = v0 + co0
                idx_v[pl.ds(bi * _CP + _L, _L)] = v1 + co1
            gs = []
            for bi in range(CH):
                gs.append(pltpu.async_copy(
                    table_hbm.at[idx_v.at[pl.ds(bi * _CP, NCAT)]],
                    blk_v.at[pl.ds(bi * T + HEAD, NCAT)], gsem))
            for bi in range(CH):
                x16 = xn_v[pl.ds(bi * F, _L)]
                r0 = bi * T
                for d in range(NVEC):
                    blk_v[r0, pl.ds(d * _L, _L)] = c_r[d]
                for i in range(F):
                    xi = lax.gather(
                        x16, jnp.full((_L, 1), i, jnp.int32),
                        lax.GatherDimensionNumbers(
                            offset_dims=(), collapsed_slice_dims=(0,),
                            start_index_map=(0,)),
                        slice_sizes=(1,),
                        mode=lax.GatherScatterMode.PROMISE_IN_BOUNDS)
                    for d in range(NVEC):
                        blk_v[r0 + 1 + i, pl.ds(d * _L, _L)] = (
                            xi * w_r[d] + b_r[d])
            nb0 = lax.min(b0 + 2 * CH, B - CH)
            fetch(nb0, raw_v, xn_v, isem)
            for g in gs:
                g.wait()
            pltpu.async_copy(blk_v, out_hbm.at[pl.ds(b0 * T, CH * T)], wsem)

        fetch(base, rawa_v, xna_v, isema)
        fetch(base + CH, rawb_v, xnb_v, isemb)

        def pair(c, carry):
            b0 = base + c * (2 * CH)

            @pl.when(c > 0)
            def _():
                drain_out(blka_v, wsema)
            do_chunk(b0, rawa_v, idxa_v, xna_v, blka_v, wsema, isema)

            @pl.when(c > 0)
            def _():
                drain_out(blkb_v, wsemb)
            do_chunk(b0 + CH, rawb_v, idxb_v, xnb_v, blkb_v, wsemb, isemb)
            return carry

        lax.fori_loop(0, NCH // 2, pair, 0)
        drain_out(blka_v, wsema)
        drain_out(blkb_v, wsemb)
        drain_in(rawa_v, xna_v, isema)
        drain_in(rawb_v, xnb_v, isemb)

    return tok(xnum_f, xcat_f, w, b, cls, coffp, table)


def kernel(x_num, x_cat, W_num, b_num, tables, cls_token):
    B, F = x_num.shape
    NCAT, V, D = tables.shape
    co = np.zeros(2 * _L, dtype=np.int32)
    co[:NCAT] = np.arange(NCAT, dtype=np.int32) * V
    out = _tokenize(B, F, NCAT, V, D,
                    x_num.reshape(-1), x_cat.astype(jnp.int32).reshape(-1),
                    W_num.reshape(D), b_num.reshape(D), cls_token.reshape(D),
                    jnp.asarray(co), tables.reshape(NCAT * V, D))
    return out.reshape(B, 1 + F + NCAT, D)


# speedup vs baseline: 1.0431x; 1.0431x over previous
import functools

import jax
import jax.numpy as jnp
import numpy as np
from jax import lax
from jax.experimental import pallas as pl
from jax.experimental.pallas import tpu as pltpu
from jax.experimental.pallas import tpu_sc as plsc

_INFO = plsc.get_sparse_core_info()
_NC, _NS, _L = _INFO.num_cores, _INFO.num_subcores, _INFO.num_lanes
_NW = _NC * _NS

_CP = 32


@functools.partial(jax.jit, static_argnums=(0, 1, 2, 3, 4))
def _tokenize(B, F, NCAT, V, D, xnum_pad, xcat_pad, w, b, cls, off, table):
    T = 1 + F + NCAT
    HEAD = 1 + F
    FP = _L
    CH = 8
    PER_W = B // _NW
    NCH = PER_W // CH
    NVEC = D // _L
    IDXN = CH * _CP
    XNN = CH * FP

    mesh = plsc.VectorSubcoreMesh(core_axis_name="c", subcore_axis_name="s")

    @functools.partial(
        pl.kernel,
        out_type=jax.ShapeDtypeStruct((B * T, D), jnp.float32),
        mesh=mesh,
        scratch_types=[
            pltpu.VMEM((IDXN,), jnp.int32),
            pltpu.VMEM((IDXN,), jnp.int32),
            pltpu.VMEM((IDXN,), jnp.int32),
            pltpu.VMEM((IDXN,), jnp.int32),
            pltpu.VMEM((XNN,), jnp.float32),
            pltpu.VMEM((XNN,), jnp.float32),
            pltpu.VMEM((CH * T, D), jnp.float32),
            pltpu.VMEM((CH * T, D), jnp.float32),
            pltpu.VMEM((D,), jnp.float32),
            pltpu.VMEM((D,), jnp.float32),
            pltpu.VMEM((D,), jnp.float32),
            pltpu.VMEM((IDXN,), jnp.int32),
            pltpu.SemaphoreType.DMA,
            pltpu.SemaphoreType.DMA,
            pltpu.SemaphoreType.DMA,
            pltpu.SemaphoreType.DMA,
            pltpu.SemaphoreType.DMA,
            pltpu.SemaphoreType.DMA,
            pltpu.SemaphoreType.DMA,
            pltpu.SemaphoreType.DMA,
            pltpu.SemaphoreType.DMA,
            pltpu.SemaphoreType.DMA,
            pltpu.SemaphoreType.DMA,
            pltpu.SemaphoreType.DMA,
        ],
    )
    def tok(xnum_hbm, xcat_hbm, w_hbm, b_hbm, cls_hbm, off_hbm, table_hbm,
            out_hbm, rawa_v, rawb_v, idxa_v, idxb_v, xna_v, xnb_v,
            blka_v, blkb_v, w_v, b_v, c_v, off_v,
            gsem0, gsem1, gsem2, gsem3, gsem4, gsem5, gsem6, gsem7,
            wsema, wsemb, isema, isemb):
        gsems = [gsem0, gsem1, gsem2, gsem3, gsem4, gsem5, gsem6, gsem7]
        wid = lax.axis_index("s") * _NC + lax.axis_index("c")
        base = wid * PER_W
        pltpu.sync_copy(w_hbm, w_v)
        pltpu.sync_copy(b_hbm, b_v)
        pltpu.sync_copy(cls_hbm, c_v)
        pltpu.sync_copy(off_hbm, off_v)
        w_r = [w_v[pl.ds(d * _L, _L)] for d in range(NVEC)]
        b_r = [b_v[pl.ds(d * _L, _L)] for d in range(NVEC)]
        c_r = [c_v[pl.ds(d * _L, _L)] for d in range(NVEC)]

        def fetch(b0, raw_v, xn_v, isem):
            pltpu.async_copy(xcat_hbm.at[pl.ds(b0 * _CP, IDXN)], raw_v, isem)
            pltpu.async_copy(xnum_hbm.at[pl.ds(b0 * FP, XNN)], xn_v, isem)

        def drain_in(raw_v, xn_v, isem):
            pltpu.make_async_copy(
                xcat_hbm.at[pl.ds(0, IDXN)], raw_v, isem).wait()
            pltpu.make_async_copy(
                xnum_hbm.at[pl.ds(0, XNN)], xn_v, isem).wait()

        def drain_out(blk_v, wsem):
            pltpu.make_async_copy(
                out_hbm.at[pl.ds(0, CH * T)], blk_v, wsem).wait()

        def do_chunk(b0, raw_v, idx_v, xn_v, blk_v, wsem, isem):
            drain_in(raw_v, xn_v, isem)
            for j in range(IDXN // _L):
                sl = pl.ds(j * _L, _L)
                idx_v[sl] = raw_v[sl] + off_v[sl]
            gs = []
            for bi in range(CH):
                gs.append(pltpu.async_copy(
                    table_hbm.at[idx_v.at[pl.ds(bi * _CP, NCAT)]],
                    blk_v.at[pl.ds(bi * T + HEAD, NCAT)], gsems[bi]))
            nb0 = lax.min(b0 + 2 * CH, B - CH)
            pltpu.async_copy(
                xcat_hbm.at[pl.ds(nb0 * _CP, IDXN)], raw_v, isem)
            for bi in range(CH):
                x16 = xn_v[pl.ds(bi * FP, _L)]
                r0 = bi * T
                for d in range(NVEC):
                    blk_v[r0, pl.ds(d * _L, _L)] = c_r[d]
                for i in range(F):
                    xi = lax.gather(
                        x16, jnp.full((_L, 1), i, jnp.int32),
                        lax.GatherDimensionNumbers(
                            offset_dims=(), collapsed_slice_dims=(0,),
                            start_index_map=(0,)),
                        slice_sizes=(1,),
                        mode=lax.GatherScatterMode.PROMISE_IN_BOUNDS)
                    for d in range(NVEC):
                        blk_v[r0 + 1 + i, pl.ds(d * _L, _L)] = (
                            xi * w_r[d] + b_r[d])
                gs[bi].wait()
                pltpu.async_copy(blk_v.at[pl.ds(r0, T)],
                                 out_hbm.at[pl.ds((b0 + bi) * T, T)], wsem)
            pltpu.async_copy(xnum_hbm.at[pl.ds(nb0 * FP, XNN)], xn_v, isem)

        fetch(base, rawa_v, xna_v, isema)
        fetch(base + CH, rawb_v, xnb_v, isemb)

        def pair(c, carry):
            b0 = base + c * (2 * CH)

            @pl.when(c > 0)
            def _():
                drain_out(blka_v, wsema)
            do_chunk(b0, rawa_v, idxa_v, xna_v, blka_v, wsema, isema)

            @pl.when(c > 0)
            def _():
                drain_out(blkb_v, wsemb)
            do_chunk(b0 + CH, rawb_v, idxb_v, xnb_v, blkb_v, wsemb, isemb)
            return carry

        lax.fori_loop(0, NCH // 2, pair, 0)
        drain_out(blka_v, wsema)
        drain_out(blkb_v, wsemb)
        drain_in(rawa_v, xna_v, isema)
        drain_in(rawb_v, xnb_v, isemb)

    return tok(xnum_pad, xcat_pad, w, b, cls, off, table)


def kernel(x_num, x_cat, W_num, b_num, tables, cls_token):
    B, F = x_num.shape
    NCAT, V, D = tables.shape
    xnum_pad = jnp.pad(x_num, ((0, 0), (0, _L - F))).reshape(-1)
    xcat_pad = jnp.pad(x_cat.astype(jnp.int32),
                       ((0, 0), (0, _CP - NCAT))).reshape(-1)
    off = jnp.asarray(np.tile(
        np.pad(np.arange(NCAT, dtype=np.int32) * V, (0, _CP - NCAT)), 8))
    out = _tokenize(B, F, NCAT, V, D, xnum_pad, xcat_pad,
                    W_num.reshape(D), b_num.reshape(D), cls_token.reshape(D),
                    off, tables.reshape(NCAT * V, D))
    return out.reshape(B, 1 + F + NCAT, D)
